# Initial kernel scaffold; baseline (speedup 1.0000x reference)
import jax
import jax.numpy as jnp
from jax.experimental import pallas as pl


def kernel(target, pos_context, neg_context, in_table, out_table):
    raise NotImplementedError("write your pallas kernel here")


# speedup vs baseline: 3.2251x; 3.2251x over previous
import functools

import jax
import jax.numpy as jnp
from jax import lax
from jax.experimental import pallas as pl
from jax.experimental.pallas import tpu as pltpu
from jax.experimental.pallas import tpu_sc as plsc

_B = 16384
_D = 64
_NPOS = 20
_NNEG = 50
_NCTX = _NPOS + _NNEG
_NC = 2
_NS = 16
_NW = _NC * _NS
_BPW = _B // _NW
_C = 16
_NCHUNK = _BPW // _C
_CW = _C * _NCTX
_IDXW = 112
_IDXR = _CW // _IDXW
_NG = 5
_ROWS_PAD = _CW + _NG * 16 - _NCTX + 6
_SC_PAD = _CW + 16

_mesh = plsc.VectorSubcoreMesh(core_axis_name="c", subcore_axis_name="s")


@functools.partial(
    pl.kernel,
    mesh=_mesh,
    out_type=jax.ShapeDtypeStruct((_B * _NCTX,), jnp.float32),
    scratch_types=[
        pltpu.VMEM((_C,), jnp.int32),
        pltpu.VMEM((_CW,), jnp.int32),
        pltpu.VMEM((_C, _D), jnp.float32),
        pltpu.VMEM((_ROWS_PAD, _D), jnp.float32),
        pltpu.VMEM((_SC_PAD,), jnp.float32),
        pltpu.SemaphoreType.DMA,
    ],
    compiler_params=pltpu.CompilerParams(
        needs_layout_passes=False, use_tc_tiling_on_sc=False),
)
def _scores_kernel(in_tab, out_tab, tgt, ctx, out,
                   idx_t, idx_c, v_buf, rows, scores_c, sem):
    wid = lax.axis_index("s") * _NC + lax.axis_index("c")
    iota = lax.iota(jnp.int32, 16)

    def chunk_body(ci, carry):
        b0 = wid * _BPW + ci * _C
        pltpu.sync_copy(tgt.at[pl.ds(b0, _C)], idx_t)
        pltpu.sync_copy(ctx.at[pl.ds(b0 * _NCTX, _CW)], idx_c)
        cps = [pltpu.async_copy(in_tab.at[idx_t], v_buf, sem)]
        for s in range(_IDXR):
            cps.append(pltpu.async_copy(
                out_tab.at[idx_c.at[pl.ds(s * _IDXW, _IDXW)]],
                rows.at[pl.ds(s * _IDXW, _IDXW)], sem))
        for cp in cps:
            cp.wait()

        def b_body(b, c2):
            vchunks = [v_buf[b, pl.ds(k * 16, 16)] for k in range(4)]
            for g in range(_NG):
                row0 = b * _NCTX + g * 16
                ridx = row0 + iota
                acc = jnp.zeros((16,), jnp.float32)
                for d in range(_D):
                    vd = vchunks[d // 16][d % 16]
                    u = plsc.load_gather(
                        rows, [ridx, jnp.full((16,), d, jnp.int32)])
                    acc = acc + u * vd
                scores_c[pl.ds(row0, 16)] = acc
            return c2

        lax.fori_loop(0, _C, b_body, 0)
        pltpu.sync_copy(scores_c.at[pl.ds(0, _CW)],
                        out.at[pl.ds(b0 * _NCTX, _CW)])
        return carry

    lax.fori_loop(0, _NCHUNK, chunk_body, 0)


def kernel(target, pos_context, neg_context, in_table, out_table):
    ctx = jnp.concatenate(
        [pos_context.astype(jnp.int32), neg_context.astype(jnp.int32)],
        axis=1)
    ctx2 = ctx.reshape(_B * _NCTX)
    scores = _scores_kernel(in_table, out_table,
                            target.astype(jnp.int32), ctx2)
    s = scores.reshape(_B, _NCTX)
    return s[:, :_NPOS], s[:, _NPOS:]


# speedup vs baseline: 3.2401x; 1.0046x over previous
import functools

import jax
import jax.numpy as jnp
from jax import lax
from jax.experimental import pallas as pl
from jax.experimental.pallas import tpu as pltpu
from jax.experimental.pallas import tpu_sc as plsc

_B = 16384
_D = 64
_V = 1000000
_NPOS = 20
_NNEG = 50
_NCTX = _NPOS + _NNEG
_NC = 2
_NS = 16
_NW = _NC * _NS
_BPW = _B // _NW
_C = 4
_NCHUNK = _BPW // _C
_CW = _C * _NCTX
_SLC = (112, 112, 56)
_NG = 5
_ROWS = 296
_CTXPAD = _BPW * _NCTX + 16
_FL = 16
_SCW = _FL * _CW

_mesh = plsc.VectorSubcoreMesh(core_axis_name="c", subcore_axis_name="s")


@functools.partial(
    pl.kernel,
    mesh=_mesh,
    out_type=jax.ShapeDtypeStruct((_B * _NCTX,), jnp.float32),
    scratch_types=[
        pltpu.VMEM((_CTXPAD,), jnp.int32),
        pltpu.VMEM((_BPW + 16,), jnp.int32),
        pltpu.VMEM((2 * (_CW + 16),), jnp.int32),
        pltpu.VMEM((32,), jnp.int32),
        pltpu.VMEM((2, _ROWS, 128), jnp.float32),
        pltpu.VMEM((2, _C, 128), jnp.float32),
        pltpu.VMEM((_SCW + 16,), jnp.float32),
        pltpu.SemaphoreType.DMA,
    ],
    compiler_params=pltpu.CompilerParams(needs_layout_passes=False),
)
def _scores_kernel(in_tab, out_tab, tgt, ctx, out,
                   ctx_all, tgt_all, gidx, vgidx, rows, vrows,
                   scores_acc, sem):
    wid = lax.axis_index("s") * _NC + lax.axis_index("c")
    iota = lax.iota(jnp.int32, 16)

    pltpu.sync_copy(tgt.at[pl.ds(wid * _BPW, _BPW)],
                    tgt_all.at[pl.ds(0, _BPW)])
    pltpu.sync_copy(ctx.at[pl.ds(wid * _BPW * _NCTX, _BPW * _NCTX)],
                    ctx_all.at[pl.ds(0, _BPW * _NCTX)])

    _GSTR = _CW + 16

    def fire(ci, buf):
        off = ci * _CW
        gb = buf * _GSTR
        for s in range(_CW // 16 + 1):
            vals = ctx_all[pl.ds(off + s * 16, 16)]
            gidx[pl.ds(gb + s * 16, 16)] = lax.shift_right_logical(vals, 1)
        tvals = tgt_all[pl.ds(ci * _C, 16)]
        vgidx[pl.ds(buf * 16, 16)] = lax.shift_right_logical(tvals, 1)
        cps = [pltpu.async_copy(in_tab.at[vgidx.at[pl.ds(buf * 16, _C)]],
                                vrows.at[buf], sem)]
        o = 0
        for n in _SLC:
            cps.append(pltpu.async_copy(
                out_tab.at[gidx.at[pl.ds(gb + o, n)]],
                rows.at[buf, pl.ds(o, n)], sem))
            o += n
        return cps

    def drain(buf):
        gb = buf * _GSTR
        pltpu.make_async_copy(in_tab.at[vgidx.at[pl.ds(buf * 16, _C)]],
                              vrows.at[buf], sem).wait()
        o = 0
        for n in _SLC:
            pltpu.make_async_copy(
                out_tab.at[gidx.at[pl.ds(gb + o, n)]],
                rows.at[buf, pl.ds(o, n)], sem).wait()
            o += n

    fire(0, 0)

    def chunk_body(ci, carry):
        buf = lax.rem(ci, 2)
        nxt = 1 - buf

        @pl.when(ci + 1 < _NCHUNK)
        def _():
            fire(ci + 1, nxt)

        drain(buf)

        tpar = lax.bitwise_and(tgt_all[pl.ds(ci * _C, 16)], jnp.int32(1))
        soff = lax.rem(ci, _FL) * _CW
        for b in range(_C):
            vcol = tpar[b] * _D
            vchunks = [vrows[buf, b, pl.ds(vcol + k * 16, 16)]
                       for k in range(4)]
            for g in range(_NG):
                row0 = b * _NCTX + g * 16
                ridx = row0 + iota
                cpar = lax.bitwise_and(
                    ctx_all[pl.ds(ci * _CW + row0, 16)], jnp.int32(1))
                ccol = cpar * _D
                acc0 = jnp.zeros((16,), jnp.float32)
                acc1 = jnp.zeros((16,), jnp.float32)
                for d in range(_D // 2):
                    u0 = plsc.load_gather(rows.at[buf], [ridx, ccol + d])
                    acc0 = acc0 + u0 * vchunks[d // 16][d % 16]
                    d2 = d + _D // 2
                    u1 = plsc.load_gather(rows.at[buf], [ridx, ccol + d2])
                    acc1 = acc1 + u1 * vchunks[d2 // 16][d2 % 16]
                scores_acc[pl.ds(soff + row0, 16)] = acc0 + acc1

        @pl.when(lax.rem(ci, _FL) == _FL - 1)
        def _():
            pltpu.sync_copy(
                scores_acc.at[pl.ds(0, _SCW)],
                out.at[pl.ds(wid * _BPW * _NCTX
                             + (ci // _FL) * _SCW, _SCW)])

        return carry

    lax.fori_loop(0, _NCHUNK, chunk_body, 0)


def kernel(target, pos_context, neg_context, in_table, out_table):
    ctx = jnp.concatenate(
        [pos_context.astype(jnp.int32), neg_context.astype(jnp.int32)],
        axis=1)
    ctx2 = ctx.reshape(_B * _NCTX)
    scores = _scores_kernel(in_table.reshape(_V // 2, 128),
                            out_table.reshape(_V // 2, 128),
                            target.astype(jnp.int32), ctx2)
    s = scores.reshape(_B, _NCTX)
    return s[:, :_NPOS], s[:, _NPOS:]


# speedup vs baseline: 3.6900x; 1.1389x over previous
import functools

import jax
import jax.numpy as jnp
from jax import lax
from jax.experimental import pallas as pl
from jax.experimental.pallas import tpu as pltpu
from jax.experimental.pallas import tpu_sc as plsc

_B = 16384
_D = 64
_NPOS = 20
_NNEG = 50
_NCTX = _NPOS + _NNEG
_NC = 2
_NS = 16
_NW = _NC * _NS
_BPW = _B // _NW
_C = 4
_NCHUNK = _BPW // _C
_CW = _C * _NCTX
_SLC = (112, 112, 56)
_FL = 16
_SCW = _FL * _CW

_mesh = plsc.VectorSubcoreMesh(core_axis_name="c", subcore_axis_name="s")


@functools.partial(
    pl.kernel,
    mesh=_mesh,
    out_type=jax.ShapeDtypeStruct((_B * _NCTX,), jnp.float32),
    scratch_types=[
        pltpu.VMEM((_BPW * _NCTX,), jnp.int32),
        pltpu.VMEM((_BPW + 16,), jnp.int32),
        pltpu.VMEM((32,), jnp.int32),
        pltpu.VMEM((2, _CW, _D), jnp.float32),
        pltpu.VMEM((2, _C, _D), jnp.float32),
        pltpu.VMEM((_SCW,), jnp.float32),
        pltpu.SemaphoreType.DMA,
    ],
    compiler_params=pltpu.CompilerParams(
        needs_layout_passes=False, use_tc_tiling_on_sc=False),
)
def _scores_kernel(in_tab, out_tab, tgt, ctx, out,
                   ctx_all, tgt_all, vgidx, rows, vrows, scores_acc, sem):
    wid = lax.axis_index("s") * _NC + lax.axis_index("c")
    iota = lax.iota(jnp.int32, 16)
    lane15 = iota == 15

    pltpu.sync_copy(tgt.at[pl.ds(wid * _BPW, _BPW)],
                    tgt_all.at[pl.ds(0, _BPW)])
    pltpu.sync_copy(ctx.at[pl.ds(wid * _BPW * _NCTX, _BPW * _NCTX)], ctx_all)

    def fire(ci, buf):
        vgidx[pl.ds(buf * 16, 16)] = tgt_all[pl.ds(ci * _C, 16)]
        cps = [pltpu.async_copy(in_tab.at[vgidx.at[pl.ds(buf * 16, _C)]],
                                vrows.at[buf], sem)]
        o = 0
        for n in _SLC:
            cps.append(pltpu.async_copy(
                out_tab.at[ctx_all.at[pl.ds(ci * _CW + o, n)]],
                rows.at[buf, pl.ds(o, n)], sem))
            o += n
        return cps

    def drain(buf, ci):
        pltpu.make_async_copy(in_tab.at[vgidx.at[pl.ds(buf * 16, _C)]],
                              vrows.at[buf], sem).wait()
        o = 0
        for n in _SLC:
            pltpu.make_async_copy(
                out_tab.at[ctx_all.at[pl.ds(ci * _CW + o, n)]],
                rows.at[buf, pl.ds(o, n)], sem).wait()
            o += n

    fire(0, 0)

    def chunk_body(ci, carry):
        buf = lax.rem(ci, 2)
        nxt = 1 - buf

        @pl.when(ci + 1 < _NCHUNK)
        def _():
            fire(ci + 1, nxt)

        drain(buf, ci)

        soff = lax.rem(ci, _FL) * _CW
        for b in range(_C):
            vchunks = [vrows[buf, b, pl.ds(k * 16, 16)] for k in range(4)]
            for jj in range(_NCTX):
                j = b * _NCTX + jj
                prod = ((rows[buf, j, pl.ds(0, 16)] * vchunks[0]
                         + rows[buf, j, pl.ds(16, 16)] * vchunks[1])
                        + (rows[buf, j, pl.ds(32, 16)] * vchunks[2]
                           + rows[buf, j, pl.ds(48, 16)] * vchunks[3]))
                cs = plsc.cumsum(prod)
                plsc.store_scatter(scores_acc,
                                   [jnp.full((16,), soff + j, jnp.int32)],
                                   cs, mask=lane15)

        @pl.when(lax.rem(ci, _FL) == _FL - 1)
        def _():
            pltpu.sync_copy(
                scores_acc,
                out.at[pl.ds(wid * _BPW * _NCTX
                             + (ci // _FL) * _SCW, _SCW)])

        return carry

    lax.fori_loop(0, _NCHUNK, chunk_body, 0)


def kernel(target, pos_context, neg_context, in_table, out_table):
    ctx = jnp.concatenate(
        [pos_context.astype(jnp.int32), neg_context.astype(jnp.int32)],
        axis=1)
    ctx2 = ctx.reshape(_B * _NCTX)
    scores = _scores_kernel(in_table, out_table,
                            target.astype(jnp.int32), ctx2)
    s = scores.reshape(_B, _NCTX)
    return s[:, :_NPOS], s[:, _NPOS:]


# speedup vs baseline: 6.0192x; 1.6312x over previous
import functools

import jax
import jax.numpy as jnp
from jax import lax
from jax.experimental import pallas as pl
from jax.experimental.pallas import tpu as pltpu
from jax.experimental.pallas import tpu_sc as plsc

_B = 16384
_D = 64
_NPOS = 20
_NNEG = 50
_NCTX = _NPOS + _NNEG
_NC = 2
_NS = 16
_NW = _NC * _NS
_BPW = _B // _NW
_C = 4
_NCHUNK = _BPW // _C
_CW = _C * _NCTX
_SLC = (112, 112, 56)
_FL = 16
_SCW = _FL * _CW
_NG = 5
_ROWS = 296

_mesh = plsc.VectorSubcoreMesh(core_axis_name="c", subcore_axis_name="s")


@functools.partial(
    pl.kernel,
    mesh=_mesh,
    out_type=jax.ShapeDtypeStruct((_B * _NCTX,), jnp.float32),
    scratch_types=[
        pltpu.VMEM((_BPW * _NCTX,), jnp.int32),
        pltpu.VMEM((_BPW + 16,), jnp.int32),
        pltpu.VMEM((32,), jnp.int32),
        pltpu.VMEM((2, _ROWS, _D), jnp.float32),
        pltpu.VMEM((2, _C, _D), jnp.float32),
        pltpu.VMEM((_C, 2 * _D), jnp.float32),
        pltpu.VMEM((_SCW + 16,), jnp.float32),
        pltpu.SemaphoreType.DMA,
    ],
    compiler_params=pltpu.CompilerParams(
        needs_layout_passes=False, use_tc_tiling_on_sc=False),
)
def _scores_kernel(in_tab, out_tab, tgt, ctx, out,
                   ctx_all, tgt_all, vgidx, rows, vrows, v2, scores_acc,
                   sem):
    wid = lax.axis_index("s") * _NC + lax.axis_index("c")
    iota = lax.iota(jnp.int32, 16)

    pltpu.sync_copy(tgt.at[pl.ds(wid * _BPW, _BPW)],
                    tgt_all.at[pl.ds(0, _BPW)])
    pltpu.sync_copy(ctx.at[pl.ds(wid * _BPW * _NCTX, _BPW * _NCTX)], ctx_all)

    def fire(ci, buf):
        vgidx[pl.ds(buf * 16, 16)] = tgt_all[pl.ds(ci * _C, 16)]
        cps = [pltpu.async_copy(in_tab.at[vgidx.at[pl.ds(buf * 16, _C)]],
                                vrows.at[buf], sem)]
        o = 0
        for n in _SLC:
            cps.append(pltpu.async_copy(
                out_tab.at[ctx_all.at[pl.ds(ci * _CW + o, n)]],
                rows.at[buf, pl.ds(o, n)], sem))
            o += n
        return cps

    def drain(buf, ci):
        pltpu.make_async_copy(in_tab.at[vgidx.at[pl.ds(buf * 16, _C)]],
                              vrows.at[buf], sem).wait()
        o = 0
        for n in _SLC:
            pltpu.make_async_copy(
                out_tab.at[ctx_all.at[pl.ds(ci * _CW + o, n)]],
                rows.at[buf, pl.ds(o, n)], sem).wait()
            o += n

    fire(0, 0)

    def chunk_body(ci, carry):
        buf = lax.rem(ci, 2)
        nxt = 1 - buf

        @pl.when(ci + 1 < _NCHUNK)
        def _():
            fire(ci + 1, nxt)

        drain(buf, ci)

        soff = lax.rem(ci, _FL) * _CW
        for b in range(_C):
            for k in range(4):
                c = vrows[buf, b, pl.ds(k * 16, 16)]
                v2[b, pl.ds(k * 16, 16)] = c
                v2[b, pl.ds(_D + k * 16, 16)] = c
        for b in range(_C):
            rbases = [(b * _NCTX + g * 16) + iota for g in range(_NG)]

            def d_body(dq, accs, b=b, rbases=rbases):
                accs = list(accs)
                for d4 in range(4):
                    d = dq * 4 + d4
                    civd = lax.bitwise_and(iota + d, jnp.int32(_D - 1))
                    vv = v2[b, pl.ds(d, 16)]
                    for g in range(_NG):
                        u = plsc.load_gather(rows.at[buf],
                                             [rbases[g], civd])
                        accs[g] = accs[g] + u * vv
                return tuple(accs)

            accs = lax.fori_loop(
                0, _D // 4, d_body,
                tuple([jnp.zeros((16,), jnp.float32)] * _NG))
            for g in range(_NG):
                scores_acc[pl.ds(soff + b * _NCTX + g * 16, 16)] = accs[g]

        @pl.when(lax.rem(ci, _FL) == _FL - 1)
        def _():
            pltpu.sync_copy(
                scores_acc.at[pl.ds(0, _SCW)],
                out.at[pl.ds(wid * _BPW * _NCTX
                             + (ci // _FL) * _SCW, _SCW)])

        return carry

    lax.fori_loop(0, _NCHUNK, chunk_body, 0)


def kernel(target, pos_context, neg_context, in_table, out_table):
    ctx = jnp.concatenate(
        [pos_context.astype(jnp.int32), neg_context.astype(jnp.int32)],
        axis=1)
    ctx2 = ctx.reshape(_B * _NCTX)
    scores = _scores_kernel(in_table, out_table,
                            target.astype(jnp.int32), ctx2)
    s = scores.reshape(_B, _NCTX)
    return s[:, :_NPOS], s[:, _NPOS:]


# speedup vs baseline: 7.5761x; 1.2587x over previous
import functools

import jax
import jax.numpy as jnp
from jax import lax
from jax.experimental import pallas as pl
from jax.experimental.pallas import tpu as pltpu
from jax.experimental.pallas import tpu_sc as plsc

_B = 16384
_D = 64
_NPOS = 20
_NNEG = 50
_NCTX = _NPOS + _NNEG
_PAD = 128
_NC = 2
_NS = 16
_NW = _NC * _NS
_BPW = _B // _NW
_C = 4
_NCHUNK = _BPW // _C
_NG = 5
_ROWS = 296
_CPW = _C * _PAD
_FL = 16
_SCW = _FL * _CPW

_mesh = plsc.VectorSubcoreMesh(core_axis_name="c", subcore_axis_name="s")


@functools.partial(
    pl.kernel,
    mesh=_mesh,
    out_type=jax.ShapeDtypeStruct((_B * _PAD,), jnp.float32),
    scratch_types=[
        pltpu.VMEM((2 * _CPW,), jnp.int32),
        pltpu.VMEM((2 * _CPW,), jnp.float32),
        pltpu.VMEM((2, _ROWS, _D), jnp.float32),
        pltpu.VMEM((_C, 2 * _D), jnp.float32),
        pltpu.VMEM((_SCW,), jnp.float32),
        pltpu.SemaphoreType.DMA,
        pltpu.SemaphoreType.DMA,
    ],
    compiler_params=pltpu.CompilerParams(
        needs_layout_passes=False, use_tc_tiling_on_sc=False),
)
def _scores_kernel(out_tab, idx, vin, out,
                   ibuf, vbuf, rows, v2, scores_acc, gsem, ssem):
    wid = lax.axis_index("s") * _NC + lax.axis_index("c")
    iota = lax.iota(jnp.int32, 16)
    base = wid * _BPW * _PAD

    def stage_pair(c, islot):
        src = pl.ds(base + c * _CPW, _CPW)
        return (pltpu.async_copy(idx.at[src],
                                 ibuf.at[pl.ds(islot * _CPW, _CPW)], ssem),
                pltpu.async_copy(vin.at[src],
                                 vbuf.at[pl.ds(islot * _CPW, _CPW)], ssem))

    def gather_cps(c, buf):
        ib = lax.rem(c, 2) * _CPW
        return [pltpu.async_copy(
            out_tab.at[ibuf.at[pl.ds(ib + b * _PAD, _NCTX)]],
            rows.at[buf, pl.ds(b * _NCTX, _NCTX)], gsem)
            for b in range(_C)]

    for cp in stage_pair(0, 0):
        cp.wait()
    stage_pair(1, 1)
    gather_cps(0, 0)

    def chunk_body(ci, carry):
        buf = lax.rem(ci, 2)
        nxt = 1 - buf

        @pl.when(ci + 1 < _NCHUNK)
        def _():
            pltpu.make_async_copy(
                idx.at[pl.ds(base + (ci + 1) * _CPW, _CPW)],
                ibuf.at[pl.ds(lax.rem(ci + 1, 2) * _CPW, _CPW)],
                ssem).wait()
            pltpu.make_async_copy(
                vin.at[pl.ds(base + (ci + 1) * _CPW, _CPW)],
                vbuf.at[pl.ds(lax.rem(ci + 1, 2) * _CPW, _CPW)],
                ssem).wait()
            gather_cps(ci + 1, nxt)

        ib = lax.rem(ci, 2) * _CPW
        for b in range(_C):
            pltpu.make_async_copy(
                out_tab.at[ibuf.at[pl.ds(ib + b * _PAD, _NCTX)]],
                rows.at[buf, pl.ds(b * _NCTX, _NCTX)], gsem).wait()

        vb = lax.rem(ci, 2) * _CPW
        for b in range(_C):
            for k in range(4):
                c = vbuf[pl.ds(vb + b * _PAD + k * 16, 16)]
                v2[b, pl.ds(k * 16, 16)] = c
                v2[b, pl.ds(_D + k * 16, 16)] = c

        @pl.when(ci + 2 < _NCHUNK)
        def _():
            stage_pair(ci + 2, lax.rem(ci, 2))

        soff = lax.rem(ci, _FL) * _CPW
        for b in range(_C):
            rbases = [(b * _NCTX + g * 16) + iota for g in range(_NG)]

            def d_body(dq, accs, b=b, rbases=rbases):
                accs = list(accs)
                for d4 in range(4):
                    d = dq * 4 + d4
                    civd = lax.bitwise_and(iota + d, jnp.int32(_D - 1))
                    vv = v2[b, pl.ds(d, 16)]
                    for g in range(_NG):
                        u = plsc.load_gather(rows.at[buf],
                                             [rbases[g], civd])
                        accs[g] = accs[g] + u * vv
                return tuple(accs)

            accs = lax.fori_loop(
                0, _D // 4, d_body,
                tuple([jnp.zeros((16,), jnp.float32)] * _NG))
            for g in range(_NG):
                scores_acc[pl.ds(soff + b * _PAD + g * 16, 16)] = accs[g]

        @pl.when(lax.rem(ci, _FL) == _FL - 1)
        def _():
            pltpu.sync_copy(
                scores_acc,
                out.at[pl.ds(base + (ci // _FL) * _SCW, _SCW)])

        return carry

    lax.fori_loop(0, _NCHUNK, chunk_body, 0)


def kernel(target, pos_context, neg_context, in_table, out_table):
    zpad = jnp.zeros((_B, _PAD - _NCTX), jnp.int32)
    idx = jnp.concatenate(
        [pos_context.astype(jnp.int32), neg_context.astype(jnp.int32),
         zpad], axis=1).reshape(_B * _PAD)
    v = jnp.take(in_table, target, axis=0)
    vp = jnp.concatenate(
        [v, jnp.zeros((_B, _PAD - _D), jnp.float32)], axis=1)
    scores = _scores_kernel(out_table, idx, vp.reshape(_B * _PAD))
    s = scores.reshape(_B, _PAD)
    return s[:, :_NPOS], s[:, _NPOS:_NCTX]
